# baseline (speedup 1.0000x reference)
import functools

import jax
import jax.numpy as jnp
from jax import lax
from jax.experimental import pallas as pl
from jax.experimental.pallas import tpu as pltpu
from jax.experimental.pallas import tpu_sc as plsc

N = 10000
NP = 10240
E = 320000
D = 128
NC = 2
NS = 16
NW = NC * NS
EPW = E // NW
K = 80
NCHUNK = 128
EPWP = NCHUNK * K
RPT = NP // NS

_mesh = plsc.VectorSubcoreMesh(core_axis_name="c", subcore_axis_name="s")


@functools.partial(
    pl.kernel,
    out_type=jax.ShapeDtypeStruct((NC, NP), jnp.float32),
    mesh=_mesh,
    scratch_types=[
        pltpu.VMEM((NCHUNK, K), jnp.int32),
        pltpu.VMEM((K,), jnp.float32),
        pltpu.VMEM_SHARED((NP,), jnp.float32),
        pltpu.SemaphoreType.DMA,
    ],
)
def _deg_partials(col2d_hbm, ones_hbm, zeros_hbm, out_hbm, colbig, onesb, acc, sem):
    c = lax.axis_index("c")
    s = lax.axis_index("s")
    wid = s * NC + c
    pltpu.sync_copy(ones_hbm, onesb)
    pltpu.sync_copy(col2d_hbm.at[pl.ds(wid * NCHUNK, NCHUNK)], colbig)

    @pl.when(s == 0)
    def _():
        pltpu.sync_copy(zeros_hbm, acc)

    plsc.subcore_barrier()

    def body(j, carry):
        pltpu.sync_copy(onesb, acc.at[colbig.at[j]], add=True)
        return carry

    lax.fori_loop(0, NCHUNK, body, 0)
    plsc.subcore_barrier()

    @pl.when(s == 0)
    def _():
        pltpu.sync_copy(acc, out_hbm.at[c])


@functools.partial(
    pl.kernel,
    out_type=jax.ShapeDtypeStruct((NC, NP, D), jnp.float32),
    mesh=_mesh,
    scratch_types=[
        pltpu.VMEM((EPWP,), jnp.int32),
        pltpu.VMEM((NCHUNK, K), jnp.int32),
        pltpu.VMEM((K, D), jnp.float32),
        pltpu.VMEM((K, D), jnp.float32),
        pltpu.VMEM_SHARED((NP, D), jnp.float32),
        pltpu.SemaphoreType.DMA,
        pltpu.SemaphoreType.DMA,
    ],
)
def _scatter_partials(y_hbm, rowp_hbm, col2d_hbm, zeros_hbm, out_hbm,
                      rowbig, colbig, g0, g1, acc, sem0, sem1):
    c = lax.axis_index("c")
    s = lax.axis_index("s")
    wid = s * NC + c
    pltpu.sync_copy(zeros_hbm.at[pl.ds(s * RPT, RPT)], acc.at[pl.ds(s * RPT, RPT)])
    pltpu.sync_copy(rowp_hbm.at[pl.ds(wid * EPWP, EPWP)], rowbig)
    pltpu.sync_copy(col2d_hbm.at[pl.ds(wid * NCHUNK, NCHUNK)], colbig)
    plsc.subcore_barrier()

    def gather(j, buf, sem):
        return pltpu.async_copy(y_hbm.at[rowbig.at[pl.ds(j * K, K)]], buf, sem)

    GRP = 8
    bufs = (g0, g1)
    sems = (sem0, sem1)

    def body(g, carry):
        j0 = g * GRP
        desc = [gather(j0, g0, sem0), gather(j0 + 1, g1, sem1)]
        for i in range(GRP):
            b = i % 2
            desc[b].wait()
            pltpu.sync_copy(bufs[b], acc.at[colbig.at[j0 + i]], add=True)
            if i + 2 < GRP:
                desc[b] = gather(j0 + i + 2, bufs[b], sems[b])
        return carry

    lax.fori_loop(0, NCHUNK // GRP, body, 0)

    plsc.subcore_barrier()
    pltpu.sync_copy(acc.at[pl.ds(s * RPT, RPT)],
                    out_hbm.at[c, pl.ds(s * RPT, RPT)])


R = 1000


def _tcb_body(degt_ref, x_ref, w_ref, y_ref, disb_ref):
    d = degt_ref[...]
    dis = lax.rsqrt(d[:, 0:1] + d[:, 1:2] + 1.0)
    xw = jnp.dot(x_ref[...], w_ref[...], preferred_element_type=jnp.float32)
    y_ref[...] = dis * xw
    disb_ref[...] = jnp.broadcast_to(dis, xw.shape)


_tcb = pl.pallas_call(
    _tcb_body,
    grid=(N // R,),
    in_specs=[
        pl.BlockSpec((R, 2), lambda i: (i, 0)),
        pl.BlockSpec((R, D), lambda i: (i, 0)),
        pl.BlockSpec((D, D), lambda i: (0, 0)),
    ],
    out_specs=[
        pl.BlockSpec((R, D), lambda i: (i, 0)),
        pl.BlockSpec((R, D), lambda i: (i, 0)),
    ],
    out_shape=[
        jax.ShapeDtypeStruct((N, D), jnp.float32),
        jax.ShapeDtypeStruct((N, D), jnp.float32),
    ],
)


def _tcd_body(a0_ref, a1_ref, y1_ref, disb_ref, b1_ref, w2_ref, y2_ref):
    dis = disb_ref[...]
    h = jnp.tanh(dis * (a0_ref[...] + a1_ref[...] + y1_ref[...]) + b1_ref[...])
    y2_ref[...] = dis * jnp.dot(h, w2_ref[...], preferred_element_type=jnp.float32)


_tcd = pl.pallas_call(
    _tcd_body,
    grid=(N // R,),
    in_specs=[
        pl.BlockSpec((R, D), lambda i: (i, 0)),
        pl.BlockSpec((R, D), lambda i: (i, 0)),
        pl.BlockSpec((R, D), lambda i: (i, 0)),
        pl.BlockSpec((R, D), lambda i: (i, 0)),
        pl.BlockSpec((1, D), lambda i: (0, 0)),
        pl.BlockSpec((D, D), lambda i: (0, 0)),
    ],
    out_specs=pl.BlockSpec((R, D), lambda i: (i, 0)),
    out_shape=jax.ShapeDtypeStruct((N, D), jnp.float32),
)


def _tcf_body(a0_ref, a1_ref, y2_ref, disb_ref, b2_ref, o_ref):
    o_ref[...] = (disb_ref[...] * (a0_ref[...] + a1_ref[...] + y2_ref[...])
                  + b2_ref[...])


_tcf = pl.pallas_call(
    _tcf_body,
    grid=(N // R,),
    in_specs=[
        pl.BlockSpec((R, D), lambda i: (i, 0)),
        pl.BlockSpec((R, D), lambda i: (i, 0)),
        pl.BlockSpec((R, D), lambda i: (i, 0)),
        pl.BlockSpec((R, D), lambda i: (i, 0)),
        pl.BlockSpec((1, D), lambda i: (0, 0)),
    ],
    out_specs=pl.BlockSpec((R, D), lambda i: (i, 0)),
    out_shape=jax.ShapeDtypeStruct((N, D), jnp.float32),
)


@jax.jit
def kernel(x, edge_index, W1, b1, W2, b2):
    row = edge_index[0].astype(jnp.int32)
    col = edge_index[1].astype(jnp.int32)
    x = x.astype(jnp.float32)

    pad_r = jnp.zeros((NW, EPWP - EPW), jnp.int32)
    pad_c = jnp.full((NW, EPWP - EPW), N, jnp.int32)
    rowp = jnp.concatenate([row.reshape(NW, EPW), pad_r], axis=1).reshape(-1)
    col2d = jnp.concatenate([col.reshape(NW, EPW), pad_c], axis=1)
    col2d = col2d.reshape(NW * NCHUNK, K)

    ones_k = jnp.ones((K,), jnp.float32)
    zeros1 = jnp.zeros((NP,), jnp.float32)
    zeros2 = jnp.zeros((NP, D), jnp.float32)

    degp = _deg_partials(col2d, ones_k, zeros1)
    degt = degp.T

    y1, disb = _tcb(degt, x, W1)
    a1 = _scatter_partials(y1, rowp, col2d, zeros2)
    y2 = _tcd(a1[0], a1[1], y1, disb, b1.reshape(1, D), W2)
    a2 = _scatter_partials(y2, rowp, col2d, zeros2)
    out = _tcf(a2[0], a2[1], y2, disb, b2.reshape(1, D))
    return out
